# Initial kernel scaffold; baseline (speedup 1.0000x reference)
import jax
import jax.numpy as jnp
from jax.experimental import pallas as pl


def kernel(x, y):
    raise NotImplementedError("write your pallas kernel here")


# speedup vs baseline: 117.2791x; 117.2791x over previous
import functools

import jax
import jax.numpy as jnp
from jax.experimental import pallas as pl
from jax.experimental.pallas import tpu as pltpu

_GAMMA = 0.1
_BIG = 1e6
_EPS = 1e-9


def _sdtw_kernel(x_ref, y_ref, out_ref, *, N, M):
    Bb = x_ref.shape[1]
    x = x_ref[:, :]

    big = x * 0.0 + _BIG
    big_row = big[:1, :]
    iota = jax.lax.broadcasted_iota(jnp.int32, (N, Bb), 0)

    def step(k, carry):
        v_km2, v_km1 = carry
        yw = y_ref[pl.ds(N + M - k, N), :]
        d = jnp.abs(x - yw)
        r0 = jnp.concatenate([big_row, v_km2[:-1, :]], axis=0)
        r0 = jnp.where((iota == 0) & (k == 2), 0.0, r0)
        r1 = jnp.concatenate([big_row, v_km1[:-1, :]], axis=0)
        r2 = v_km1
        a0 = r0 * (-1.0 / _GAMMA)
        a1 = r1 * (-1.0 / _GAMMA)
        a2 = r2 * (-1.0 / _GAMMA)
        rmax = jnp.maximum(jnp.maximum(a0, a1), a2)
        rsum = jnp.exp(a0 - rmax) + jnp.exp(a1 - rmax) + jnp.exp(a2 - rmax)
        softmin = -_GAMMA * (jnp.log(rsum + _EPS) + rmax)
        mask = (iota >= k - M - 1) & (iota <= k - 2)
        v_k = jnp.where(mask, d + softmin, _BIG)
        return (v_km1, v_k)

    _, v_last = jax.lax.fori_loop(2, N + M + 1, step, (big, big))
    out_ref[0, 0, :] = v_last[N - 1, :]


def kernel(x, y):
    B, N = x.shape
    M = y.shape[1]
    x_t = x.T
    y_rev = y[:, ::-1].T
    pad_left = N - 1
    total = pad_left + M + (N - 1)
    padded = ((total + 7) // 8) * 8
    y_pad = jnp.zeros((padded, B), jnp.float32).at[pad_left:pad_left + M].set(y_rev)

    Bb = 128
    NB = B // Bb
    out = pl.pallas_call(
        functools.partial(_sdtw_kernel, N=N, M=M),
        grid=(NB,),
        in_specs=[
            pl.BlockSpec((N, Bb), lambda i: (0, i)),
            pl.BlockSpec((padded, Bb), lambda i: (0, i)),
        ],
        out_specs=pl.BlockSpec((1, 1, Bb), lambda i: (i, 0, 0)),
        out_shape=jax.ShapeDtypeStruct((NB, 1, Bb), jnp.float32),
        compiler_params=pltpu.CompilerParams(dimension_semantics=("parallel",)),
    )(x_t, y_pad)
    loss = out.reshape(B) / (N + M)
    return loss.mean()


# speedup vs baseline: 133.4907x; 1.1382x over previous
import functools
import math

import jax
import jax.numpy as jnp
from jax.experimental import pallas as pl
from jax.experimental.pallas import tpu as pltpu

_GAMMA = 0.1
_BIG = 1e6
_EPS = 1e-9
_C1 = -math.log2(math.e) / _GAMMA
_C2 = -_GAMMA * math.log(2.0)


def _sdtw_kernel(x_ref, y_ref, out_ref, *, N, M):
    x = x_ref[:, :]
    big = x * 0.0 + _BIG
    big_row = big[:1, :]
    zero_row = big_row * 0.0

    def body(k, v_km2, v_km1, r0_row):
        yw = y_ref[pl.ds(N + M - k, N), :]
        d = jnp.abs(x - yw)
        r0 = jnp.concatenate([r0_row, v_km2[:-1, :]], axis=0)
        r1 = jnp.concatenate([big_row, v_km1[:-1, :]], axis=0)
        r2 = v_km1
        b0 = r0 * _C1
        b1 = r1 * _C1
        b2 = r2 * _C1
        bmax = jnp.maximum(jnp.maximum(b0, b1), b2)
        rsum = jnp.exp2(b0 - bmax) + jnp.exp2(b1 - bmax) + jnp.exp2(b2 - bmax)
        softmin = _C2 * (jnp.log2(rsum + _EPS) + bmax)
        return d + softmin

    v2 = body(2, big, big, zero_row)

    def step(k, carry):
        v_km2, v_km1 = carry
        return (v_km1, body(k, v_km2, v_km1, big_row))

    _, v_last = jax.lax.fori_loop(3, N + M + 1, step, (big, v2))
    out_ref[0, 0, :] = v_last[N - 1, :]


def kernel(x, y):
    B, N = x.shape
    M = y.shape[1]
    x_t = x.T
    y_rev = y[:, ::-1].T
    pad_left = N - 1
    total = pad_left + M + (N - 1)
    padded = ((total + 7) // 8) * 8
    y_pad = jnp.zeros((padded, B), jnp.float32).at[pad_left:pad_left + M].set(y_rev)

    Bb = 128
    NB = B // Bb
    out = pl.pallas_call(
        functools.partial(_sdtw_kernel, N=N, M=M),
        grid=(NB,),
        in_specs=[
            pl.BlockSpec((N, Bb), lambda i: (0, i)),
            pl.BlockSpec((padded, Bb), lambda i: (0, i)),
        ],
        out_specs=pl.BlockSpec((1, 1, Bb), lambda i: (i, 0, 0)),
        out_shape=jax.ShapeDtypeStruct((NB, 1, Bb), jnp.float32),
        compiler_params=pltpu.CompilerParams(dimension_semantics=("parallel",)),
    )(x_t, y_pad)
    loss = out.reshape(B) / (N + M)
    return loss.mean()


# speedup vs baseline: 152.9220x; 1.1456x over previous
import functools
import math

import jax
import jax.numpy as jnp
from jax.experimental import pallas as pl
from jax.experimental.pallas import tpu as pltpu

_GAMMA = 0.1
_BIG = 1e6
_EPS = 1e-9
_C1 = -math.log2(math.e) / _GAMMA
_C2 = -_GAMMA * math.log(2.0)


def _sdtw_kernel(x_ref, y_ref, out_ref, *, N, M):
    x = x_ref[:, :]
    big = x * 0.0 + _BIG
    big_row = big[:1, :]
    zero_row = big_row * 0.0

    def body(k, v_km2, v_km1, r0_row):
        yw = y_ref[pl.ds(N + M - k, N), :]
        d = jnp.abs(x - yw)
        r0 = jnp.concatenate([r0_row, v_km2[:-1, :]], axis=0)
        r1 = jnp.concatenate([big_row, v_km1[:-1, :]], axis=0)
        r2 = v_km1
        b0 = r0 * _C1
        b1 = r1 * _C1
        b2 = r2 * _C1
        bmax = jnp.maximum(jnp.maximum(b0, b1), b2)
        rsum = jnp.exp2(b0 - bmax) + jnp.exp2(b1 - bmax) + jnp.exp2(b2 - bmax)
        softmin = _C2 * (jnp.log2(rsum) + bmax)
        return d + softmin

    v2 = body(2, big, big, zero_row)

    def step(k, carry):
        v_km2, v_km1 = carry
        return (v_km1, body(k, v_km2, v_km1, big_row))

    _, v_last = jax.lax.fori_loop(3, N + M + 1, step, (big, v2), unroll=2)
    out_ref[0, 0, :] = v_last[N - 1, :]


def kernel(x, y):
    B, N = x.shape
    M = y.shape[1]
    x_t = x.T
    y_rev = y[:, ::-1].T
    pad_left = N - 1
    total = pad_left + M + (N - 1)
    padded = ((total + 7) // 8) * 8
    y_pad = jnp.zeros((padded, B), jnp.float32).at[pad_left:pad_left + M].set(y_rev)

    Bb = 128
    NB = B // Bb
    out = pl.pallas_call(
        functools.partial(_sdtw_kernel, N=N, M=M),
        grid=(NB,),
        in_specs=[
            pl.BlockSpec((N, Bb), lambda i: (0, i)),
            pl.BlockSpec((padded, Bb), lambda i: (0, i)),
        ],
        out_specs=pl.BlockSpec((1, 1, Bb), lambda i: (i, 0, 0)),
        out_shape=jax.ShapeDtypeStruct((NB, 1, Bb), jnp.float32),
        compiler_params=pltpu.CompilerParams(dimension_semantics=("parallel",)),
    )(x_t, y_pad)
    loss = out.reshape(B) / (N + M)
    return loss.mean()


# speedup vs baseline: 163.6185x; 1.0699x over previous
import functools
import math

import jax
import jax.numpy as jnp
from jax.experimental import pallas as pl
from jax.experimental.pallas import tpu as pltpu

_GAMMA = 0.1
_BIG = 1e6
_EPS = 1e-9
_C1 = -math.log2(math.e) / _GAMMA
_C2 = -_GAMMA * math.log(2.0)


def _sdtw_kernel(x_ref, y_ref, out_ref, *, N, M):
    x = x_ref[:, :]
    big = x * 0.0 + _BIG
    big_row = big[:1, :]
    zero_row = big_row * 0.0

    def body(k, v_km2, v_km1, r0_row):
        yw = y_ref[pl.ds(N + M - k, N), :]
        d = jnp.abs(x - yw)
        r0 = jnp.concatenate([r0_row, v_km2[:-1, :]], axis=0)
        r1 = jnp.concatenate([big_row, v_km1[:-1, :]], axis=0)
        r2 = v_km1
        b0 = r0 * _C1
        b1 = r1 * _C1
        b2 = r2 * _C1
        bmax = jnp.maximum(jnp.maximum(b0, b1), b2)
        rsum = jnp.exp2(b0 - bmax) + jnp.exp2(b1 - bmax) + jnp.exp2(b2 - bmax)
        softmin = _C2 * (jnp.log2(rsum) + bmax)
        return d + softmin

    v2 = body(2, big, big, zero_row)

    def step(k, carry):
        v_km2, v_km1 = carry
        return (v_km1, body(k, v_km2, v_km1, big_row))

    _, v_last = jax.lax.fori_loop(3, N + M + 1, step, (big, v2), unroll=4)
    out_ref[0, 0, :] = v_last[N - 1, :]


def kernel(x, y):
    B, N = x.shape
    M = y.shape[1]
    x_t = x.T
    y_rev = y[:, ::-1].T
    pad_left = N - 1
    total = pad_left + M + (N - 1)
    padded = ((total + 7) // 8) * 8
    y_pad = jnp.zeros((padded, B), jnp.float32).at[pad_left:pad_left + M].set(y_rev)

    Bb = 128
    NB = B // Bb
    out = pl.pallas_call(
        functools.partial(_sdtw_kernel, N=N, M=M),
        grid=(NB,),
        in_specs=[
            pl.BlockSpec((N, Bb), lambda i: (0, i)),
            pl.BlockSpec((padded, Bb), lambda i: (0, i)),
        ],
        out_specs=pl.BlockSpec((1, 1, Bb), lambda i: (i, 0, 0)),
        out_shape=jax.ShapeDtypeStruct((NB, 1, Bb), jnp.float32),
        compiler_params=pltpu.CompilerParams(dimension_semantics=("parallel",)),
    )(x_t, y_pad)
    loss = out.reshape(B) / (N + M)
    return loss.mean()


# speedup vs baseline: 167.3435x; 1.0228x over previous
import functools
import math

import jax
import jax.numpy as jnp
from jax.experimental import pallas as pl
from jax.experimental.pallas import tpu as pltpu

_GAMMA = 0.1
_BIG = 1e6
_EPS = 1e-9
_C1 = -math.log2(math.e) / _GAMMA
_C2 = -_GAMMA * math.log(2.0)


def _sdtw_kernel(x_ref, y_ref, out_ref, *, N, M):
    x = x_ref[:, :]
    big = x * 0.0 + _BIG
    big_row = big[:1, :]
    zero_row = big_row * 0.0

    def body(k, v_km2, v_km1, r0_row):
        yw = y_ref[pl.ds(N + M - k, N), :]
        d = jnp.abs(x - yw)
        r0 = jnp.concatenate([r0_row, v_km2[:-1, :]], axis=0)
        r1 = jnp.concatenate([big_row, v_km1[:-1, :]], axis=0)
        r2 = v_km1
        b0 = r0 * _C1
        b1 = r1 * _C1
        b2 = r2 * _C1
        bmax = jnp.maximum(jnp.maximum(b0, b1), b2)
        rsum = jnp.exp2(b0 - bmax) + jnp.exp2(b1 - bmax) + jnp.exp2(b2 - bmax)
        softmin = _C2 * (jnp.log2(rsum) + bmax)
        return d + softmin

    v2 = body(2, big, big, zero_row)

    def step(k, carry):
        v_km2, v_km1 = carry
        return (v_km1, body(k, v_km2, v_km1, big_row))

    _, v_last = jax.lax.fori_loop(3, N + M + 1, step, (big, v2), unroll=8)
    out_ref[0, 0, :] = v_last[N - 1, :]


def kernel(x, y):
    B, N = x.shape
    M = y.shape[1]
    x_t = x.T
    y_rev = y[:, ::-1].T
    pad_left = N - 1
    total = pad_left + M + (N - 1)
    padded = ((total + 7) // 8) * 8
    y_pad = jnp.zeros((padded, B), jnp.float32).at[pad_left:pad_left + M].set(y_rev)

    Bb = 128
    NB = B // Bb
    out = pl.pallas_call(
        functools.partial(_sdtw_kernel, N=N, M=M),
        grid=(NB,),
        in_specs=[
            pl.BlockSpec((N, Bb), lambda i: (0, i)),
            pl.BlockSpec((padded, Bb), lambda i: (0, i)),
        ],
        out_specs=pl.BlockSpec((1, 1, Bb), lambda i: (i, 0, 0)),
        out_shape=jax.ShapeDtypeStruct((NB, 1, Bb), jnp.float32),
        compiler_params=pltpu.CompilerParams(dimension_semantics=("parallel",)),
    )(x_t, y_pad)
    loss = out.reshape(B) / (N + M)
    return loss.mean()


# speedup vs baseline: 233.3097x; 1.3942x over previous
import functools
import math

import jax
import jax.numpy as jnp
from jax.experimental import pallas as pl
from jax.experimental.pallas import tpu as pltpu

_GAMMA = 0.1
_BIG = 1e6
_C1 = -math.log2(math.e) / _GAMMA
_C2 = -_GAMMA * math.log(2.0)


def _sdtw_kernel(x_ref, y_ref, out_ref, *, N, M):
    x = x_ref[:, :]
    big = x * 0.0 + _BIG
    big_row = big[:1, :]
    zero_row = big_row * 0.0
    H = N // 2

    def make_body(xs, off):
        L = xs.shape[0]

        def body(k, v_km2, v_km1, r0_row, r1_row):
            yw = y_ref[pl.ds(off + N + M - k, L), :]
            d = jnp.abs(xs - yw)
            r0 = jnp.concatenate([r0_row, v_km2[:-1, :]], axis=0)
            r1 = jnp.concatenate([r1_row, v_km1[:-1, :]], axis=0)
            r2 = v_km1
            b0 = r0 * _C1
            b1 = r1 * _C1
            b2 = r2 * _C1
            bmax = jnp.maximum(jnp.maximum(b0, b1), b2)
            rsum = jnp.exp2(b0 - bmax) + jnp.exp2(b1 - bmax) + jnp.exp2(b2 - bmax)
            return d + _C2 * (jnp.log2(rsum) + bmax)

        return body

    body_lo = make_body(x[:H, :], 0)
    body_full = make_body(x, 0)
    body_hi = make_body(x[H:, :], H)
    big_h = big[:H, :]

    v2 = body_lo(2, big_h, big_h, zero_row, big_row)

    def step_lo(k, carry):
        a, b = carry
        return (b, body_lo(k, a, b, big_row, big_row))

    a, b = jax.lax.fori_loop(3, H + 2, step_lo, (big_h, v2), unroll=8)

    A = jnp.concatenate([a, big_h], axis=0)
    B = jnp.concatenate([b, big_h], axis=0)

    def step_full(k, carry):
        a, b = carry
        return (b, body_full(k, a, b, big_row, big_row))

    A, B = jax.lax.fori_loop(H + 2, N + H + 1, step_full, (A, B), unroll=8)

    k3 = N + H + 1
    row_a = A[H - 1:H, :]
    row_b = B[H - 1:H, :]
    a3 = A[H:, :]
    b3 = B[H:, :]
    v0 = body_hi(k3, a3, b3, row_a, row_b)
    v1 = body_hi(k3 + 1, b3, v0, row_b, big_row)

    def step_hi(k, carry):
        a, b = carry
        return (b, body_hi(k, a, b, big_row, big_row))

    _, v_last = jax.lax.fori_loop(k3 + 2, N + M + 1, step_hi, (v0, v1), unroll=8)
    out_ref[0, 0, :] = v_last[H - 1, :]


def kernel(x, y):
    B, N = x.shape
    M = y.shape[1]
    x_t = x.T
    y_rev = y[:, ::-1].T
    pad_left = N - 1
    total = pad_left + M + (N - 1)
    padded = ((total + 7) // 8) * 8
    y_pad = jnp.zeros((padded, B), jnp.float32).at[pad_left:pad_left + M].set(y_rev)

    Bb = 128
    NB = B // Bb
    out = pl.pallas_call(
        functools.partial(_sdtw_kernel, N=N, M=M),
        grid=(NB,),
        in_specs=[
            pl.BlockSpec((N, Bb), lambda i: (0, i)),
            pl.BlockSpec((padded, Bb), lambda i: (0, i)),
        ],
        out_specs=pl.BlockSpec((1, 1, Bb), lambda i: (i, 0, 0)),
        out_shape=jax.ShapeDtypeStruct((NB, 1, Bb), jnp.float32),
        compiler_params=pltpu.CompilerParams(dimension_semantics=("parallel",)),
    )(x_t, y_pad)
    loss = out.reshape(B) / (N + M)
    return loss.mean()


# speedup vs baseline: 253.6497x; 1.0872x over previous
import functools
import math

import jax
import jax.numpy as jnp
from jax.experimental import pallas as pl
from jax.experimental.pallas import tpu as pltpu

_GAMMA = 0.1
_BIG = 1e6
_C1 = -math.log2(math.e) / _GAMMA
_C2 = -_GAMMA * math.log(2.0)


def _sdtw_kernel(x_ref, y_ref, out_ref, *, N, M):
    x = x_ref[:, :]
    big = x * 0.0 + _BIG
    big_row = big[:1, :]
    zero_row = big_row * 0.0
    H = N // 2

    def make_body(xs, off):
        L = xs.shape[0]

        def body(k, v_km2, v_km1, r0_row, r1_row):
            yw = y_ref[pl.ds(off + N + M - k, L), :]
            d = jnp.abs(xs - yw)
            r0 = jnp.concatenate([r0_row, v_km2[:-1, :]], axis=0)
            r1 = jnp.concatenate([r1_row, v_km1[:-1, :]], axis=0)
            r2 = v_km1
            b0 = r0 * _C1
            b1 = r1 * _C1
            b2 = r2 * _C1
            bmax = jnp.maximum(jnp.maximum(b0, b1), b2)
            rsum = jnp.exp2(b0 - bmax) + jnp.exp2(b1 - bmax) + jnp.exp2(b2 - bmax)
            return d + _C2 * (jnp.log2(rsum) + bmax)

        return body

    Q = N // 4

    def run(body, k_lo, k_hi, a, b, unroll=8):
        def step(k, carry):
            a, b = carry
            return (b, body(k, a, b, big_row, big_row))

        return jax.lax.fori_loop(k_lo, k_hi, step, (a, b), unroll=unroll)

    def shrink(body, k_first, a, b, cut):
        row_a = a[cut - 1:cut, :]
        row_b = b[cut - 1:cut, :]
        v0 = body(k_first, a[cut:, :], b[cut:, :], row_a, row_b)
        v1 = body(k_first + 1, b[cut:, :], v0, row_b, big_row)
        return v0, v1

    body_q0 = make_body(x[:Q, :], 0)
    big_q = big[:Q, :]
    v2 = body_q0(2, big_q, big_q, zero_row, big_row)
    a, b = run(body_q0, 3, Q + 2, big_q, v2)

    body_h0 = make_body(x[:H, :], 0)
    a, b = run(body_h0, Q + 2, H + 2,
               jnp.concatenate([a, big_q], axis=0),
               jnp.concatenate([b, big_q], axis=0))

    body_full = make_body(x, 0)
    big_h = big[:H, :]
    a, b = run(body_full, H + 2, N + H + 1,
               jnp.concatenate([a, big_h], axis=0),
               jnp.concatenate([b, big_h], axis=0))

    body_hi = make_body(x[H:, :], H)
    v0, v1 = shrink(body_hi, N + H + 1, a, b, H)
    a, b = run(body_hi, N + H + 3, N + M - Q + 1, v0, v1)

    body_q1 = make_body(x[N - Q:, :], N - Q)
    v0, v1 = shrink(body_q1, N + M - Q + 1, a, b, Q)
    _, v_last = run(body_q1, N + M - Q + 3, N + M + 1, v0, v1)
    out_ref[0, 0, :] = v_last[Q - 1, :]


def kernel(x, y):
    B, N = x.shape
    M = y.shape[1]
    x_t = x.T
    y_rev = y[:, ::-1].T
    pad_left = N - 1
    total = pad_left + M + (N - 1)
    padded = ((total + 7) // 8) * 8
    y_pad = jnp.zeros((padded, B), jnp.float32).at[pad_left:pad_left + M].set(y_rev)

    Bb = 128
    NB = B // Bb
    out = pl.pallas_call(
        functools.partial(_sdtw_kernel, N=N, M=M),
        grid=(NB,),
        in_specs=[
            pl.BlockSpec((N, Bb), lambda i: (0, i)),
            pl.BlockSpec((padded, Bb), lambda i: (0, i)),
        ],
        out_specs=pl.BlockSpec((1, 1, Bb), lambda i: (i, 0, 0)),
        out_shape=jax.ShapeDtypeStruct((NB, 1, Bb), jnp.float32),
        compiler_params=pltpu.CompilerParams(dimension_semantics=("parallel",)),
    )(x_t, y_pad)
    loss = out.reshape(B) / (N + M)
    return loss.mean()
